# Initial kernel scaffold; baseline (speedup 1.0000x reference)
import jax
import jax.numpy as jnp
from jax.experimental import pallas as pl


def kernel(features, edge_index, W, b):
    raise NotImplementedError("write your pallas kernel here")


# speedup vs baseline: 5.3942x; 5.3942x over previous
import functools

import jax
import jax.numpy as jnp
from jax import lax
from jax.experimental import pallas as pl
from jax.experimental.pallas import tpu as pltpu
from jax.experimental.pallas import tpu_sc as plsc

N_NODES = 10000
N_EDGES = 320000
D = 128

NC = 2
NS = 16
N_TILES = NC * NS

EDGES_PER_TILE = N_EDGES // N_TILES
BATCH = 80
N_BATCHES = EDGES_PER_TILE // BATCH
N_PAD = 10240
ROWS_PER_TILE = N_PAD // NS
CHUNK = 128
N_CHUNKS = ROWS_PER_TILE // CHUNK


@functools.partial(
    pl.kernel,
    mesh=plsc.VectorSubcoreMesh(core_axis_name="c", subcore_axis_name="s"),
    out_type=jax.ShapeDtypeStruct((NC, N_PAD, D), jnp.float32),
    scratch_types=[
        pltpu.VMEM((BATCH,), jnp.int32),
        pltpu.VMEM((BATCH,), jnp.int32),
        pltpu.VMEM((BATCH, D), jnp.float32),
        pltpu.VMEM((CHUNK, D), jnp.float32),
        pltpu.VMEM_SHARED((N_PAD, D), jnp.float32),
        pltpu.SemaphoreType.DMA,
    ],
)
def _aggregate(src_hbm, dst_hbm, feat_hbm, out_hbm,
               src_v, dst_v, rows_v, buf_v, acc_sh, sem):
    c = lax.axis_index("c")
    s = lax.axis_index("s")

    zeros16 = jnp.zeros((16,), jnp.float32)

    def zero_body(i, _):
        r = i // (D // 16)
        col = (i % (D // 16)) * 16
        buf_v[r, pl.ds(col, 16)] = zeros16
        return 0

    lax.fori_loop(0, CHUNK * (D // 16), zero_body, 0)

    row0 = s * ROWS_PER_TILE
    for j in range(N_CHUNKS):
        pltpu.sync_copy(buf_v, acc_sh.at[pl.ds(row0 + j * CHUNK, CHUNK)])
    plsc.subcore_barrier()

    edge0 = (c * NS + s) * EDGES_PER_TILE

    def edge_body(i, _):
        base = edge0 + i * BATCH
        pltpu.sync_copy(src_hbm.at[pl.ds(base, BATCH)], src_v)
        pltpu.sync_copy(dst_hbm.at[pl.ds(base, BATCH)], dst_v)
        pltpu.async_copy(feat_hbm.at[src_v], rows_v, sem).wait()
        pltpu.sync_copy(rows_v, acc_sh.at[dst_v], add=True)
        return 0

    lax.fori_loop(0, N_BATCHES, edge_body, 0)
    plsc.subcore_barrier()

    for j in range(N_CHUNKS):
        r = row0 + j * CHUNK
        pltpu.sync_copy(acc_sh.at[pl.ds(r, CHUNK)], buf_v)
        pltpu.sync_copy(buf_v, out_hbm.at[c, pl.ds(r, CHUNK)])


def _linear_body(h0_ref, h1_ref, w_ref, b_ref, o_ref):
    h = h0_ref[...] + h1_ref[...]
    y = jnp.dot(h, w_ref[...], preferred_element_type=jnp.float32)
    o_ref[...] = jnp.maximum(y + b_ref[...], 0.0)


_ROW_BLK = 1000

_linear = pl.pallas_call(
    _linear_body,
    grid=(N_NODES // _ROW_BLK,),
    in_specs=[
        pl.BlockSpec((_ROW_BLK, D), lambda i: (i, 0)),
        pl.BlockSpec((_ROW_BLK, D), lambda i: (i, 0)),
        pl.BlockSpec((D, D), lambda i: (0, 0)),
        pl.BlockSpec((1, D), lambda i: (0, 0)),
    ],
    out_specs=pl.BlockSpec((_ROW_BLK, D), lambda i: (i, 0)),
    out_shape=jax.ShapeDtypeStruct((N_NODES, D), jnp.float32),
)


def kernel(features, edge_index, W, b):
    ei = edge_index.astype(jnp.int32)
    hp = _aggregate(ei[0], ei[1], features)
    return _linear(hp[0, :N_NODES], hp[1, :N_NODES], W, b.reshape(1, D))


# speedup vs baseline: 10.8213x; 2.0061x over previous
import functools

import jax
import jax.numpy as jnp
from jax import lax
from jax.experimental import pallas as pl
from jax.experimental.pallas import tpu as pltpu
from jax.experimental.pallas import tpu_sc as plsc

N_NODES = 10000
N_EDGES = 320000
D = 128

NC = 2
NS = 16
N_TILES = NC * NS

EDGES_PER_TILE = N_EDGES // N_TILES
BATCH = 125
N_BATCHES = EDGES_PER_TILE // BATCH
IDX_CHUNK = 16
N_ICHUNKS = N_BATCHES // IDX_CHUNK
PAIRS_PER_CHUNK = IDX_CHUNK // 2
N_PAD = 10240
ROWS_PER_TILE = N_PAD // NS
CHUNK = 80
N_CHUNKS = ROWS_PER_TILE // CHUNK


@functools.partial(
    pl.kernel,
    mesh=plsc.VectorSubcoreMesh(core_axis_name="c", subcore_axis_name="s"),
    out_type=jax.ShapeDtypeStruct((NC, N_PAD, D), jnp.float32),
    scratch_types=[
        pltpu.VMEM((IDX_CHUNK, BATCH), jnp.int32),
        pltpu.VMEM((IDX_CHUNK, BATCH), jnp.int32),
        pltpu.VMEM((2, BATCH, D), jnp.float32),
        pltpu.VMEM_SHARED((N_PAD, D), jnp.float32),
        pltpu.SemaphoreType.DMA,
        pltpu.SemaphoreType.DMA,
    ],
)
def _aggregate(src_hbm, dst_hbm, feat_hbm, out_hbm,
               src_v, dst_v, rows_v, acc_sh, sem0, sem1):
    c = lax.axis_index("c")
    s = lax.axis_index("s")
    w = c * NS + s

    zeros16 = jnp.zeros((16,), jnp.float32)

    def zero_body(i, _):
        r = i // (D // 16)
        col = (i % (D // 16)) * 16
        rows_v[0, r, pl.ds(col, 16)] = zeros16
        return 0

    lax.fori_loop(0, CHUNK * (D // 16), zero_body, 0)

    row0 = s * ROWS_PER_TILE
    zsrc = rows_v.at[0].at[pl.ds(0, CHUNK)]
    for j in range(N_CHUNKS):
        pltpu.sync_copy(zsrc, acc_sh.at[pl.ds(row0 + j * CHUNK, CHUNK)])

    plsc.subcore_barrier()

    sems = (sem0, sem1)

    def gather_start(i, slot):
        return pltpu.async_copy(feat_hbm.at[src_v.at[i]], rows_v.at[slot],
                                sems[slot])

    def gather_wait(i, slot):
        pltpu.make_async_copy(feat_hbm.at[src_v.at[i]], rows_v.at[slot],
                              sems[slot]).wait()

    for ch in range(N_ICHUNKS):
        pltpu.sync_copy(src_hbm.at[w, pl.ds(ch * IDX_CHUNK, IDX_CHUNK)], src_v)
        pltpu.sync_copy(dst_hbm.at[w, pl.ds(ch * IDX_CHUNK, IDX_CHUNK)], dst_v)
        gather_start(0, 0)
        gather_start(1, 1)

        def pair_body(k, _):
            for slot in range(2):
                i = 2 * k + slot
                gather_wait(i, slot)
                pltpu.sync_copy(rows_v.at[slot], acc_sh.at[dst_v.at[i]],
                                add=True)

                @pl.when(k < PAIRS_PER_CHUNK - 1)
                def _():
                    gather_start(i + 2, slot)
            return 0

        lax.fori_loop(0, PAIRS_PER_CHUNK, pair_body, 0)
    plsc.subcore_barrier()

    stage = rows_v.at[0].at[pl.ds(0, CHUNK)]
    for j in range(N_CHUNKS):
        r = row0 + j * CHUNK
        pltpu.sync_copy(acc_sh.at[pl.ds(r, CHUNK)], stage)
        pltpu.sync_copy(stage, out_hbm.at[c, pl.ds(r, CHUNK)])


def _linear_body(h0_ref, h1_ref, w_ref, b_ref, o_ref):
    h = h0_ref[...] + h1_ref[...]
    y = jnp.dot(h, w_ref[...], preferred_element_type=jnp.float32)
    o_ref[...] = jnp.maximum(y + b_ref[...], 0.0)


_ROW_BLK = 1000

_linear = pl.pallas_call(
    _linear_body,
    grid=(N_NODES // _ROW_BLK,),
    in_specs=[
        pl.BlockSpec((_ROW_BLK, D), lambda i: (i, 0)),
        pl.BlockSpec((_ROW_BLK, D), lambda i: (i, 0)),
        pl.BlockSpec((D, D), lambda i: (0, 0)),
        pl.BlockSpec((1, D), lambda i: (0, 0)),
    ],
    out_specs=pl.BlockSpec((_ROW_BLK, D), lambda i: (i, 0)),
    out_shape=jax.ShapeDtypeStruct((N_NODES, D), jnp.float32),
)


def kernel(features, edge_index, W, b):
    ei = edge_index.astype(jnp.int32).reshape(2, N_TILES, N_BATCHES, BATCH)
    hp = _aggregate(ei[0], ei[1], features)
    return _linear(hp[0, :N_NODES], hp[1, :N_NODES], W, b.reshape(1, D))


# speedup vs baseline: 12.5563x; 1.1603x over previous
import functools

import jax
import jax.numpy as jnp
from jax import lax
from jax.experimental import pallas as pl
from jax.experimental.pallas import tpu as pltpu
from jax.experimental.pallas import tpu_sc as plsc

N_NODES = 10000
N_EDGES = 320000
D = 128

NC = 2
NS = 16
N_TILES = NC * NS

EDGES_PER_TILE = N_EDGES // N_TILES
BATCH = 125
N_BATCHES = EDGES_PER_TILE // BATCH
IDX_CHUNK = 16
N_ICHUNKS = N_BATCHES // IDX_CHUNK
PAIRS_PER_CHUNK = IDX_CHUNK // 2
N_PAD = 10240
ROWS_PER_TILE = N_PAD // NS
CHUNK = 80
N_CHUNKS = ROWS_PER_TILE // CHUNK


@functools.partial(
    pl.kernel,
    mesh=plsc.VectorSubcoreMesh(core_axis_name="c", subcore_axis_name="s"),
    out_type=jax.ShapeDtypeStruct((NC, N_PAD, D), jnp.float32),
    scratch_types=[
        pltpu.VMEM((IDX_CHUNK, BATCH), jnp.int32),
        pltpu.VMEM((IDX_CHUNK, BATCH), jnp.int32),
        pltpu.VMEM((2, BATCH, D), jnp.float32),
        pltpu.VMEM_SHARED((N_PAD, D), jnp.float32),
        pltpu.SemaphoreType.DMA,
        pltpu.SemaphoreType.DMA,
    ],
)
def _aggregate(ei_hbm, feat_hbm, out_hbm,
               src_v, dst_v, rows_v, acc_sh, sem0, sem1):
    c = lax.axis_index("c")
    s = lax.axis_index("s")
    w = c * NS + s

    zeros16 = jnp.zeros((16,), jnp.float32)

    def zero_body(i, _):
        r = i // (D // 16)
        col = (i % (D // 16)) * 16
        rows_v[0, r, pl.ds(col, 16)] = zeros16
        return 0

    lax.fori_loop(0, CHUNK * (D // 16), zero_body, 0)

    row0 = s * ROWS_PER_TILE
    zsrc = rows_v.at[0].at[pl.ds(0, CHUNK)]
    for j in range(N_CHUNKS):
        pltpu.sync_copy(zsrc, acc_sh.at[pl.ds(row0 + j * CHUNK, CHUNK)])

    plsc.subcore_barrier()

    sems = (sem0, sem1)

    def gather_start(i, slot):
        return pltpu.async_copy(feat_hbm.at[src_v.at[i]], rows_v.at[slot],
                                sems[slot])

    def gather_wait(i, slot):
        pltpu.make_async_copy(feat_hbm.at[src_v.at[i]], rows_v.at[slot],
                              sems[slot]).wait()

    for ch in range(N_ICHUNKS):
        pltpu.sync_copy(ei_hbm.at[0, w, pl.ds(ch * IDX_CHUNK, IDX_CHUNK)], src_v)
        pltpu.sync_copy(ei_hbm.at[1, w, pl.ds(ch * IDX_CHUNK, IDX_CHUNK)], dst_v)
        gather_start(0, 0)
        gather_start(1, 1)

        def pair_body(k, _):
            for slot in range(2):
                i = 2 * k + slot
                gather_wait(i, slot)
                pltpu.sync_copy(rows_v.at[slot], acc_sh.at[dst_v.at[i]],
                                add=True)

                @pl.when(k < PAIRS_PER_CHUNK - 1)
                def _():
                    gather_start(i + 2, slot)
            return 0

        lax.fori_loop(0, PAIRS_PER_CHUNK, pair_body, 0)
    plsc.subcore_barrier()

    stage = rows_v.at[0].at[pl.ds(0, CHUNK)]
    for j in range(N_CHUNKS):
        r = row0 + j * CHUNK
        pltpu.sync_copy(acc_sh.at[pl.ds(r, CHUNK)], stage)
        pltpu.sync_copy(stage, out_hbm.at[c, pl.ds(r, CHUNK)])


def _linear_body(h0_ref, h1_ref, w_ref, b_ref, o_ref):
    h = h0_ref[0] + h1_ref[0]
    y = jnp.dot(h, w_ref[...], preferred_element_type=jnp.float32)
    o_ref[...] = jnp.maximum(y + b_ref[...], 0.0)


_ROW_BLK = 1000

_linear = pl.pallas_call(
    _linear_body,
    grid=(N_NODES // _ROW_BLK,),
    in_specs=[
        pl.BlockSpec((1, _ROW_BLK, D), lambda i: (0, i, 0)),
        pl.BlockSpec((1, _ROW_BLK, D), lambda i: (1, i, 0)),
        pl.BlockSpec((D, D), lambda i: (0, 0)),
        pl.BlockSpec((1, D), lambda i: (0, 0)),
    ],
    out_specs=pl.BlockSpec((_ROW_BLK, D), lambda i: (i, 0)),
    out_shape=jax.ShapeDtypeStruct((N_NODES, D), jnp.float32),
)


def kernel(features, edge_index, W, b):
    ei = edge_index.astype(jnp.int32).reshape(2, N_TILES, N_BATCHES, BATCH)
    hp = _aggregate(ei, features)
    return _linear(hp, hp, W, b.reshape(1, D))
